# Initial kernel scaffold; baseline (speedup 1.0000x reference)
import functools

import jax
import jax.numpy as jnp
from jax import lax
from jax.experimental import pallas as pl
from jax.experimental.pallas import tpu as pltpu
from jax.experimental.pallas import tpu_sc as plsc

_N = 10000
_E = 320000
_D = 128
_KAPPA = 0.95

_NC = 2
_NS = 16
_NW = _NC * _NS
_EPW = _E // _NW
_CHUNK = 80
_NCHUNK = _EPW // _CHUNK
_RPS = _N // _NS
_ZROWS = 125

_mesh = plsc.VectorSubcoreMesh(core_axis_name="c", subcore_axis_name="s")


@functools.partial(
    pl.kernel,
    out_type=jax.ShapeDtypeStruct((_NC, _N, 16), jnp.float32),
    mesh=_mesh,
    scratch_types=[
        pltpu.VMEM((_CHUNK,), jnp.int32),
        pltpu.VMEM((_CHUNK, 16), jnp.float32),
        pltpu.VMEM((_ZROWS, 16), jnp.float32),
        pltpu.VMEM_SHARED((_N, 16), jnp.float32),
    ],
)
def _sc_degree(dst_hbm, out_hbm, idx_v, ones_v, z_v, acc_sh):
    c = lax.axis_index("c")
    s = lax.axis_index("s")

    @pl.loop(0, _CHUNK)
    def _(r):
        ones_v.at[pl.ds(r, 1), pl.ds(0, 16)][...] = jnp.ones((1, 16), jnp.float32)

    @pl.loop(0, _ZROWS)
    def _(r):
        z_v.at[pl.ds(r, 1), pl.ds(0, 16)][...] = jnp.zeros((1, 16), jnp.float32)

    @pl.loop(0, _RPS // _ZROWS)
    def _(k):
        pltpu.sync_copy(z_v, acc_sh.at[pl.ds(s * _RPS + k * _ZROWS, _ZROWS)])

    plsc.subcore_barrier()
    w = c * _NS + s

    @pl.loop(0, _NCHUNK)
    def _(i):
        off = w * _EPW + i * _CHUNK
        pltpu.sync_copy(dst_hbm.at[pl.ds(off, _CHUNK)], idx_v)
        pltpu.sync_copy(ones_v, acc_sh.at[idx_v], add=True)

    plsc.subcore_barrier()
    pltpu.sync_copy(acc_sh.at[pl.ds(s * _RPS, _RPS)],
                    out_hbm.at[c, pl.ds(s * _RPS, _RPS)])


@functools.partial(
    pl.kernel,
    out_type=jax.ShapeDtypeStruct((_NC, _N, _D), jnp.float32),
    mesh=_mesh,
    scratch_types=[
        pltpu.VMEM((_CHUNK,), jnp.int32),
        pltpu.VMEM((_CHUNK,), jnp.int32),
        pltpu.VMEM((_CHUNK, _D), jnp.float32),
        pltpu.VMEM((_ZROWS, _D), jnp.float32),
        pltpu.VMEM_SHARED((_N, _D), jnp.float32),
        pltpu.SemaphoreType.DMA,
    ],
)
def _sc_messages(g_hbm, src_hbm, dst_hbm, out_hbm,
                 sidx_v, didx_v, rows_v, z_v, acc_sh, sem):
    c = lax.axis_index("c")
    s = lax.axis_index("s")

    @pl.loop(0, _ZROWS)
    def _(r):
        @pl.loop(0, _D, step=16)
        def _(cc):
            z_v.at[pl.ds(r, 1), pl.ds(cc, 16)][...] = jnp.zeros((1, 16), jnp.float32)

    @pl.loop(0, _RPS // _ZROWS)
    def _(k):
        pltpu.sync_copy(z_v, acc_sh.at[pl.ds(s * _RPS + k * _ZROWS, _ZROWS)])

    plsc.subcore_barrier()
    w = c * _NS + s

    @pl.loop(0, _NCHUNK)
    def _(i):
        off = w * _EPW + i * _CHUNK
        pltpu.sync_copy(src_hbm.at[pl.ds(off, _CHUNK)], sidx_v)
        pltpu.sync_copy(dst_hbm.at[pl.ds(off, _CHUNK)], didx_v)
        pltpu.async_copy(g_hbm.at[sidx_v], rows_v, sem).wait()
        pltpu.sync_copy(rows_v, acc_sh.at[didx_v], add=True)

    plsc.subcore_barrier()
    pltpu.sync_copy(acc_sh.at[pl.ds(s * _RPS, _RPS)],
                    out_hbm.at[c, pl.ds(s * _RPS, _RPS)])


_BLK = 2000
_GRID = _N // _BLK


def _tc_prep_body(wc_ref, wm_ref, emb_ref, feat_ref, h_ref, dense_ref):
    wc = wc_ref[...]
    rs = jnp.sum(jnp.abs(wc), axis=1, keepdims=True)
    scale = jnp.where(rs > _KAPPA, _KAPPA / rs, 1.0)
    wproj = wc * scale
    h_ref[...] = jnp.dot(emb_ref[...], wproj.T,
                         preferred_element_type=jnp.float32,
                         precision=lax.Precision.HIGHEST)
    dense_ref[...] = jnp.dot(feat_ref[...], wm_ref[...].T,
                             preferred_element_type=jnp.float32,
                             precision=lax.Precision.HIGHEST)


def _tc_prep(W_conv, W_mlp, emb, feat):
    return pl.pallas_call(
        _tc_prep_body,
        grid=(_GRID,),
        in_specs=[
            pl.BlockSpec((_D, _D), lambda i: (0, 0)),
            pl.BlockSpec((_D, _D), lambda i: (0, 0)),
            pl.BlockSpec((_BLK, _D), lambda i: (i, 0)),
            pl.BlockSpec((_BLK, _D), lambda i: (i, 0)),
        ],
        out_specs=[
            pl.BlockSpec((_BLK, _D), lambda i: (i, 0)),
            pl.BlockSpec((_BLK, _D), lambda i: (i, 0)),
        ],
        out_shape=[
            jax.ShapeDtypeStruct((_N, _D), jnp.float32),
            jax.ShapeDtypeStruct((_N, _D), jnp.float32),
        ],
    )(W_conv, W_mlp, emb, feat)


def _tc_scale_body(h_ref, degp_ref, g_ref):
    deg = degp_ref[0, :, 0:1] + degp_ref[1, :, 0:1] + 1.0
    g_ref[...] = h_ref[...] * lax.rsqrt(deg)


def _tc_scale(h, degp):
    return pl.pallas_call(
        _tc_scale_body,
        grid=(_GRID,),
        in_specs=[
            pl.BlockSpec((_BLK, _D), lambda i: (i, 0)),
            pl.BlockSpec((_NC, _BLK, 16), lambda i: (0, i, 0)),
        ],
        out_specs=pl.BlockSpec((_BLK, _D), lambda i: (i, 0)),
        out_shape=jax.ShapeDtypeStruct((_N, _D), jnp.float32),
    )(h, degp)


def _tc_final_body(p_ref, g_ref, dense_ref, degp_ref, o_ref):
    deg = degp_ref[0, :, 0:1] + degp_ref[1, :, 0:1] + 1.0
    dinv = lax.rsqrt(deg)
    acc = p_ref[0] + p_ref[1] + g_ref[...]
    o_ref[...] = jnp.maximum(acc * dinv + dense_ref[...], 0.0)


def _tc_final(parts, g, dense, degp):
    return pl.pallas_call(
        _tc_final_body,
        grid=(_GRID,),
        in_specs=[
            pl.BlockSpec((_NC, _BLK, _D), lambda i: (0, i, 0)),
            pl.BlockSpec((_BLK, _D), lambda i: (i, 0)),
            pl.BlockSpec((_BLK, _D), lambda i: (i, 0)),
            pl.BlockSpec((_NC, _BLK, 16), lambda i: (0, i, 0)),
        ],
        out_specs=pl.BlockSpec((_BLK, _D), lambda i: (i, 0)),
        out_shape=jax.ShapeDtypeStruct((_N, _D), jnp.float32),
    )(parts, g, dense, degp)


def kernel(features, sparse_adj, W_conv, W_mlp, embeddings):
    src = sparse_adj[0]
    dst = sparse_adj[1]
    degp = _sc_degree(dst)
    h, dense = _tc_prep(W_conv, W_mlp, embeddings, features)
    g = _tc_scale(h, degp)
    parts = _sc_messages(g, src, dst)
    return _tc_final(parts, g, dense, degp)


# speedup vs baseline: 17.8223x; 17.8223x over previous
import functools

import jax
import jax.numpy as jnp
from jax import lax
from jax.experimental import pallas as pl
from jax.experimental.pallas import tpu as pltpu
from jax.experimental.pallas import tpu_sc as plsc

_N = 10000
_E = 320000
_D = 128
_KAPPA = 0.95

_NC = 2
_NS = 16
_NW = _NC * _NS
_EPW = _E // _NW
_CHUNK = 80
_NCHUNK = _EPW // _CHUNK
_NP = 10240
_RPS = _NP // _NS
_ZROWS = 128

_mesh = plsc.VectorSubcoreMesh(core_axis_name="c", subcore_axis_name="s")


@functools.partial(
    pl.kernel,
    out_type=jax.ShapeDtypeStruct((_NC, _NP, 16), jnp.float32),
    mesh=_mesh,
    scratch_types=[
        pltpu.VMEM((_CHUNK,), jnp.int32),
        pltpu.VMEM((_CHUNK, 16), jnp.float32),
        pltpu.VMEM((_ZROWS, 16), jnp.float32),
        pltpu.VMEM_SHARED((_NP, 16), jnp.float32),
    ],
)
def _sc_degree(dst_hbm, out_hbm, idx_v, ones_v, z_v, acc_sh):
    c = lax.axis_index("c")
    s = lax.axis_index("s")

    @pl.loop(0, _CHUNK)
    def _(r):
        ones_v.at[pl.ds(r, 1), pl.ds(0, 16)][...] = jnp.ones((1, 16), jnp.float32)

    @pl.loop(0, _ZROWS)
    def _(r):
        z_v.at[pl.ds(r, 1), pl.ds(0, 16)][...] = jnp.zeros((1, 16), jnp.float32)

    @pl.loop(0, _RPS // _ZROWS)
    def _(k):
        pltpu.sync_copy(z_v, acc_sh.at[pl.ds(s * _RPS + k * _ZROWS, _ZROWS)])

    plsc.subcore_barrier()
    w = c * _NS + s

    @pl.loop(0, _NCHUNK)
    def _(i):
        off = w * _EPW + i * _CHUNK
        pltpu.sync_copy(dst_hbm.at[pl.ds(off, _CHUNK)], idx_v)
        pltpu.sync_copy(ones_v, acc_sh.at[idx_v], add=True)

    plsc.subcore_barrier()
    pltpu.sync_copy(acc_sh.at[pl.ds(s * _RPS, _RPS)],
                    out_hbm.at[c, pl.ds(s * _RPS, _RPS)])


@functools.partial(
    pl.kernel,
    out_type=jax.ShapeDtypeStruct((_NC, _NP, _D), jnp.float32),
    mesh=_mesh,
    scratch_types=[
        pltpu.VMEM((_CHUNK,), jnp.int32),
        pltpu.VMEM((_CHUNK,), jnp.int32),
        pltpu.VMEM((_CHUNK, _D), jnp.float32),
        pltpu.VMEM((_ZROWS, _D), jnp.float32),
        pltpu.VMEM_SHARED((_NP, _D), jnp.float32),
        pltpu.SemaphoreType.DMA,
    ],
)
def _sc_messages(g_hbm, src_hbm, dst_hbm, out_hbm,
                 sidx_v, didx_v, rows_v, z_v, acc_sh, sem):
    c = lax.axis_index("c")
    s = lax.axis_index("s")

    @pl.loop(0, _ZROWS)
    def _(r):
        @pl.loop(0, _D, step=16)
        def _(cc):
            z_v.at[pl.ds(r, 1), pl.ds(cc, 16)][...] = jnp.zeros((1, 16), jnp.float32)

    @pl.loop(0, _RPS // _ZROWS)
    def _(k):
        pltpu.sync_copy(z_v, acc_sh.at[pl.ds(s * _RPS + k * _ZROWS, _ZROWS)])

    plsc.subcore_barrier()
    w = c * _NS + s

    @pl.loop(0, _NCHUNK)
    def _(i):
        off = w * _EPW + i * _CHUNK
        pltpu.sync_copy(src_hbm.at[pl.ds(off, _CHUNK)], sidx_v)
        pltpu.sync_copy(dst_hbm.at[pl.ds(off, _CHUNK)], didx_v)
        pltpu.async_copy(g_hbm.at[sidx_v], rows_v, sem).wait()
        pltpu.sync_copy(rows_v, acc_sh.at[didx_v], add=True)

    plsc.subcore_barrier()
    pltpu.sync_copy(acc_sh.at[pl.ds(s * _RPS, _RPS)],
                    out_hbm.at[c, pl.ds(s * _RPS, _RPS)])


_BLK = 2000
_GRID = _N // _BLK


def _tc_prep_body(wc_ref, wm_ref, emb_ref, feat_ref, h_ref, dense_ref):
    wc = wc_ref[...]
    rs = jnp.sum(jnp.abs(wc), axis=1, keepdims=True)
    scale = jnp.where(rs > _KAPPA, _KAPPA / rs, 1.0)
    wproj = wc * scale
    h_ref[...] = jnp.dot(emb_ref[...], wproj.T,
                         preferred_element_type=jnp.float32,
                         precision=lax.Precision.HIGHEST)
    dense_ref[...] = jnp.dot(feat_ref[...], wm_ref[...].T,
                             preferred_element_type=jnp.float32,
                             precision=lax.Precision.HIGHEST)


def _tc_prep(W_conv, W_mlp, emb, feat):
    return pl.pallas_call(
        _tc_prep_body,
        grid=(_GRID,),
        in_specs=[
            pl.BlockSpec((_D, _D), lambda i: (0, 0)),
            pl.BlockSpec((_D, _D), lambda i: (0, 0)),
            pl.BlockSpec((_BLK, _D), lambda i: (i, 0)),
            pl.BlockSpec((_BLK, _D), lambda i: (i, 0)),
        ],
        out_specs=[
            pl.BlockSpec((_BLK, _D), lambda i: (i, 0)),
            pl.BlockSpec((_BLK, _D), lambda i: (i, 0)),
        ],
        out_shape=[
            jax.ShapeDtypeStruct((_N, _D), jnp.float32),
            jax.ShapeDtypeStruct((_N, _D), jnp.float32),
        ],
    )(W_conv, W_mlp, emb, feat)


def _tc_scale_body(h_ref, degp_ref, g_ref):
    deg = degp_ref[0, :, 0:1] + degp_ref[1, :, 0:1] + 1.0
    g_ref[...] = h_ref[...] * lax.rsqrt(deg)


def _tc_scale(h, degp):
    return pl.pallas_call(
        _tc_scale_body,
        grid=(_GRID,),
        in_specs=[
            pl.BlockSpec((_BLK, _D), lambda i: (i, 0)),
            pl.BlockSpec((_NC, _BLK, 16), lambda i: (0, i, 0)),
        ],
        out_specs=pl.BlockSpec((_BLK, _D), lambda i: (i, 0)),
        out_shape=jax.ShapeDtypeStruct((_N, _D), jnp.float32),
    )(h, degp)


def _tc_final_body(p_ref, g_ref, dense_ref, degp_ref, o_ref):
    deg = degp_ref[0, :, 0:1] + degp_ref[1, :, 0:1] + 1.0
    dinv = lax.rsqrt(deg)
    acc = p_ref[0] + p_ref[1] + g_ref[...]
    o_ref[...] = jnp.maximum(acc * dinv + dense_ref[...], 0.0)


def _tc_final(parts, g, dense, degp):
    return pl.pallas_call(
        _tc_final_body,
        grid=(_GRID,),
        in_specs=[
            pl.BlockSpec((_NC, _BLK, _D), lambda i: (0, i, 0)),
            pl.BlockSpec((_BLK, _D), lambda i: (i, 0)),
            pl.BlockSpec((_BLK, _D), lambda i: (i, 0)),
            pl.BlockSpec((_NC, _BLK, 16), lambda i: (0, i, 0)),
        ],
        out_specs=pl.BlockSpec((_BLK, _D), lambda i: (i, 0)),
        out_shape=jax.ShapeDtypeStruct((_N, _D), jnp.float32),
    )(parts, g, dense, degp)


def kernel(features, sparse_adj, W_conv, W_mlp, embeddings):
    src = sparse_adj[0]
    dst = sparse_adj[1]
    degp = _sc_degree(dst)
    h, dense = _tc_prep(W_conv, W_mlp, embeddings, features)
    g = _tc_scale(h, degp)
    parts = _sc_messages(g, src, dst)
    return _tc_final(parts, g, dense, degp)
